# baseline (speedup 1.0000x reference)
import functools

import jax
import jax.numpy as jnp
from jax import lax
from jax.experimental import pallas as pl
from jax.experimental.pallas import tpu as pltpu
from jax.experimental.pallas import tpu_sc as plsc

NF = 8
VOCAB = 100000
BATCH = 16384
EMBED = 64
LANES = 16

NC = 2
NS = 16
NW = NC * NS

RCHUNK = 200
NCHUNKS = (VOCAB // 2) // RCHUNK
KMAX = -(-NCHUNKS // NW)

BPW = BATCH // NW
CHUNK = 64
ROUNDS = BPW // CHUNK
GROUPS = CHUNK // LANES


def _relayout_body(w0, w1, w2, w3, w4, w5, w6, w7,
                   o0, o1, o2, o3, o4, o5, o6, o7,
                   bufA, bufB):
    tables = [w0, w1, w2, w3, w4, w5, w6, w7]
    outs = [o0, o1, o2, o3, o4, o5, o6, o7]
    wid = lax.axis_index("s") * NC + lax.axis_index("c")

    for t in range(NF):
        def chunk_body(k, carry):
            c = wid + k * NW

            @pl.when(c < NCHUNKS)
            def _():
                pltpu.sync_copy(tables[t].at[pl.ds(c * 2 * RCHUNK, 2 * RCHUNK)],
                                bufA)

                def repack(j, cr):
                    for h in range(2):
                        for cc in range(EMBED // LANES):
                            bufB[j, pl.ds(h * EMBED + cc * LANES, LANES)] = (
                                bufA[2 * j + h, pl.ds(cc * LANES, LANES)])
                    return cr

                lax.fori_loop(0, RCHUNK, repack, 0)
                pltpu.sync_copy(bufB, outs[t].at[c])
            return carry

        lax.fori_loop(0, KMAX, chunk_body, 0)


_relayout = functools.partial(
    pl.kernel,
    mesh=plsc.VectorSubcoreMesh(core_axis_name="c", subcore_axis_name="s"),
    out_type=tuple(
        jax.ShapeDtypeStruct((NCHUNKS, RCHUNK, 2 * EMBED), jnp.float32)
        for _ in range(NF)),
    scratch_types=[
        pltpu.VMEM((2 * RCHUNK, EMBED), jnp.float32),
        pltpu.VMEM((RCHUNK, 2 * EMBED), jnp.float32),
    ],
)(_relayout_body)


def _gather_body(i0, i1, i2, i3, i4, i5, i6, i7,
                 w0, w1, w2, w3, w4, w5, w6, w7,
                 out, idx_v, idxj_v, buf, outb, sem):
    idxs = [i0, i1, i2, i3, i4, i5, i6, i7]
    tables = [w0, w1, w2, w3, w4, w5, w6, w7]
    wid = lax.axis_index("s") * NC + lax.axis_index("c")
    base = wid * BPW

    for f in range(NF):
        pltpu.sync_copy(idxs[f].at[pl.ds(base, BPW)], idx_v.at[f])

    def shift(i, carry):
        for f in range(NF):
            v = idx_v[f, pl.ds(i * LANES, LANES)]
            idxj_v[f, pl.ds(i * LANES, LANES)] = v >> 1
        return carry

    lax.fori_loop(0, BPW // LANES, shift, 0)

    def round_body(r, carry):
        cps = [
            pltpu.async_copy(
                tables[f].at[idxj_v.at[f, pl.ds(r * CHUNK, CHUNK)]],
                buf.at[f], sem)
            for f in range(NF)
        ]
        for cp in cps:
            cp.wait()

        def sum_group(g, carry2):
            vecs = [idx_v[f, pl.ds(r * CHUNK + g * LANES, LANES)]
                    for f in range(NF)]
            for jj in range(LANES):
                i = g * LANES + jj
                starts = [(vecs[f][jj] & 1) * EMBED for f in range(NF)]
                for c in range(EMBED // LANES):
                    acc = buf[0, i, pl.ds(starts[0] + c * LANES, LANES)]
                    for f in range(1, NF):
                        acc = acc + buf[f, i, pl.ds(starts[f] + c * LANES, LANES)]
                    outb[i, pl.ds(c * LANES, LANES)] = acc
            return carry2

        lax.fori_loop(0, GROUPS, sum_group, 0)
        pltpu.sync_copy(outb, out.at[pl.ds(base + r * CHUNK, CHUNK)])
        return carry

    lax.fori_loop(0, ROUNDS, round_body, 0)


_poly_gather = functools.partial(
    pl.kernel,
    mesh=plsc.VectorSubcoreMesh(core_axis_name="c", subcore_axis_name="s"),
    out_type=jax.ShapeDtypeStruct((BATCH, EMBED), jnp.float32),
    scratch_types=[
        pltpu.VMEM((NF, BPW), jnp.int32),
        pltpu.VMEM((NF, BPW), jnp.int32),
        pltpu.VMEM((NF, CHUNK, 2 * EMBED), jnp.float32),
        pltpu.VMEM((CHUNK, EMBED), jnp.float32),
        pltpu.SemaphoreType.DMA,
    ],
)(_gather_body)


@jax.jit
def kernel(idx_0, idx_1, idx_2, idx_3, idx_4, idx_5, idx_6, idx_7,
           W_0, W_1, W_2, W_3, W_4, W_5, W_6, W_7):
    packed = _relayout(W_0, W_1, W_2, W_3, W_4, W_5, W_6, W_7)
    tables = [p.reshape(VOCAB // 2, 2 * EMBED) for p in packed]
    return _poly_gather(idx_0, idx_1, idx_2, idx_3, idx_4, idx_5, idx_6, idx_7,
                        *tables)


# speedup vs baseline: 1.6049x; 1.6049x over previous
import functools

import jax
import jax.numpy as jnp
from jax import lax
from jax.experimental import pallas as pl
from jax.experimental.pallas import tpu as pltpu
from jax.experimental.pallas import tpu_sc as plsc

NF = 8
VOCAB = 100000
BATCH = 16384
EMBED = 64
LANES = 16

NC = 2
NS = 16
NW = NC * NS

TBLK = 1000
TSTEPS = VOCAB // TBLK

BPW = BATCH // NW
CHUNK = 64
ROUNDS = BPW // CHUNK


def _widen_body(*refs):
    ins, outs = refs[:NF], refs[NF:]
    for f in range(NF):
        x = ins[f][...]
        outs[f][...] = jnp.concatenate([x, jnp.zeros_like(x)], axis=1)


_widen = pl.pallas_call(
    _widen_body,
    grid=(TSTEPS,),
    in_specs=[pl.BlockSpec((TBLK, EMBED), lambda i: (i, 0))
              for _ in range(NF)],
    out_specs=[pl.BlockSpec((TBLK, 2 * EMBED), lambda i: (i, 0))
               for _ in range(NF)],
    out_shape=[jax.ShapeDtypeStruct((VOCAB, 2 * EMBED), jnp.float32)
               for _ in range(NF)],
)


def _gather_body(i0, i1, i2, i3, i4, i5, i6, i7,
                 w0, w1, w2, w3, w4, w5, w6, w7,
                 out, idx_v, buf, outb, sem):
    idxs = [i0, i1, i2, i3, i4, i5, i6, i7]
    tables = [w0, w1, w2, w3, w4, w5, w6, w7]
    wid = lax.axis_index("s") * NC + lax.axis_index("c")
    base = wid * BPW

    for f in range(NF):
        pltpu.sync_copy(idxs[f].at[pl.ds(base, BPW)], idx_v.at[f])

    def round_body(r, carry):
        cps = [
            pltpu.async_copy(
                tables[f].at[idx_v.at[f, pl.ds(r * CHUNK, CHUNK)]],
                buf.at[f], sem)
            for f in range(NF)
        ]
        for cp in cps:
            cp.wait()

        def sum_row(i, carry2):
            for c in range(EMBED // LANES):
                acc = buf[0, i, pl.ds(c * LANES, LANES)]
                for f in range(1, NF):
                    acc = acc + buf[f, i, pl.ds(c * LANES, LANES)]
                outb[i, pl.ds(c * LANES, LANES)] = acc
            return carry2

        lax.fori_loop(0, CHUNK, sum_row, 0)
        pltpu.sync_copy(outb, out.at[pl.ds(base + r * CHUNK, CHUNK)])
        return carry

    lax.fori_loop(0, ROUNDS, round_body, 0)


_poly_gather = functools.partial(
    pl.kernel,
    mesh=plsc.VectorSubcoreMesh(core_axis_name="c", subcore_axis_name="s"),
    out_type=jax.ShapeDtypeStruct((BATCH, EMBED), jnp.float32),
    scratch_types=[
        pltpu.VMEM((NF, BPW), jnp.int32),
        pltpu.VMEM((NF, CHUNK, 2 * EMBED), jnp.float32),
        pltpu.VMEM((CHUNK, EMBED), jnp.float32),
        pltpu.SemaphoreType.DMA,
    ],
)(_gather_body)


@jax.jit
def kernel(idx_0, idx_1, idx_2, idx_3, idx_4, idx_5, idx_6, idx_7,
           W_0, W_1, W_2, W_3, W_4, W_5, W_6, W_7):
    wide = _widen(W_0, W_1, W_2, W_3, W_4, W_5, W_6, W_7)
    return _poly_gather(idx_0, idx_1, idx_2, idx_3, idx_4, idx_5, idx_6, idx_7,
                        *wide)
